# Initial kernel scaffold; baseline (speedup 1.0000x reference)
import functools

import jax
import jax.numpy as jnp
from jax import lax
from jax.experimental import pallas as pl
from jax.experimental.pallas import tpu as pltpu
from jax.experimental.pallas import tpu_sc as plsc

L = 16
NC = 2
NS = 16
NW = NC * NS
RPW = 320
NPAD = NW * RPW
D = 16
ATT = 16
F = 128
NEG = -1e30


def _iota16():
    return lax.iota(jnp.int32, L)


def _mm_body(x_ref, wkq_ref, w_ref, kq_ref, nh_ref):
    x = x_ref[...]
    kq_ref[...] = jnp.dot(x, wkq_ref[...], preferred_element_type=jnp.float32)
    nh_ref[...] = jnp.dot(x, w_ref[...], preferred_element_type=jnp.float32)


def _matmuls(x_pad, wkq, w):
    blk = 512
    grid = NPAD // blk
    return pl.pallas_call(
        _mm_body,
        grid=(grid,),
        in_specs=[
            pl.BlockSpec((blk, F), lambda i: (i, 0)),
            pl.BlockSpec((F, 2 * ATT), lambda i: (0, 0)),
            pl.BlockSpec((F, F), lambda i: (0, 0)),
        ],
        out_specs=[
            pl.BlockSpec((blk, 2 * ATT), lambda i: (i, 0)),
            pl.BlockSpec((blk, F), lambda i: (i, 0)),
        ],
        out_shape=[
            jax.ShapeDtypeStruct((NPAD, 2 * ATT), jnp.float32),
            jax.ShapeDtypeStruct((NPAD, F), jnp.float32),
        ],
    )(x_pad, wkq, w)


def _bn_body(n_valid, x_ref, g_ref, b_ref, o_ref):
    x = x_ref[...]
    rows = lax.broadcasted_iota(jnp.int32, x.shape, 0)
    xm = jnp.where(rows < n_valid, x, 0.0)
    s = jnp.sum(xm, axis=0, keepdims=True)
    ss = jnp.sum(xm * xm, axis=0, keepdims=True)
    mean = s / n_valid
    var = ss / n_valid - mean * mean
    inv = lax.rsqrt(var + 1e-5)
    y = g_ref[...] * (x - mean) * inv + b_ref[...]
    o_ref[...] = jnp.maximum(y, 0.0)


def _batchnorm_relu(fh0, gamma, beta, n_valid):
    return pl.pallas_call(
        functools.partial(_bn_body, n_valid),
        out_shape=jax.ShapeDtypeStruct((NPAD, F), jnp.float32),
    )(fh0, gamma.reshape(1, F), beta.reshape(1, F))


GRP = 8
NGRP = RPW // GRP


def _sc1_body(n1, rfl_hbm, key_hbm, q_hbm, nh_hbm, out_hbm,
              rfl_v, key_v, nhs_v, qg, nhg, og, att_v, sem_q, sem_n):
    wid = lax.axis_index("s") * NC + lax.axis_index("c")
    base = wid * RPW
    pltpu.sync_copy(rfl_hbm.at[pl.ds(base * D, RPW * D)], rfl_v)
    pltpu.sync_copy(key_hbm.at[pl.ds(base, RPW)], key_v)
    pltpu.sync_copy(nh_hbm.at[pl.ds(base, RPW)], nhs_v)

    @pl.loop(0, NGRP)
    def _group(g):
        idx = rfl_v.at[pl.ds(g * (GRP * D), GRP * D)]
        cq = pltpu.async_copy(q_hbm.at[idx], qg, sem_q)
        cn = pltpu.async_copy(nh_hbm.at[idx], nhg, sem_n)
        cq.wait()
        cn.wait()

        @pl.loop(0, GRP)
        def _row(r8):
            row = g * GRP + r8
            kb = [jnp.full((L,), key_v[row, l]) for l in range(ATT)]
            m16 = r8 * D + _iota16()
            recep = jnp.zeros((L,), jnp.float32)
            for l in range(ATT):
                col = plsc.load_gather(qg, [m16, jnp.full((L,), l, jnp.int32)])
                recep = recep + col * kb[l]
            rfrow = rfl_v[pl.ds(row * D, D)]
            recep = jnp.where(rfrow == n1, NEG, recep)
            mx = jnp.max(recep)
            e = jnp.exp(recep - mx)
            att = e / jnp.sum(e)
            att_v[...] = att
            acc = [nhs_v[row, pl.ds(c * L, L)] for c in range(F // L)]
            for j in range(D):
                wb = jnp.full((L,), att_v[j])
                for c in range(F // L):
                    acc[c] = acc[c] + wb * nhg[r8 * D + j, pl.ds(c * L, L)]
            for c in range(F // L):
                og[r8, pl.ds(c * L, L)] = acc[c]

        pltpu.sync_copy(og, out_hbm.at[pl.ds(base + g * GRP, GRP)])


def _sc_stage1(rfl, key, query, nh, n1):
    mesh = plsc.VectorSubcoreMesh(core_axis_name="c", subcore_axis_name="s")
    return pl.kernel(
        functools.partial(_sc1_body, n1),
        out_type=jax.ShapeDtypeStruct((NPAD, F), jnp.float32),
        mesh=mesh,
        scratch_types=[
            pltpu.VMEM((RPW * D,), jnp.int32),
            pltpu.VMEM((RPW, ATT), jnp.float32),
            pltpu.VMEM((RPW, F), jnp.float32),
            pltpu.VMEM((GRP * D, ATT), jnp.float32),
            pltpu.VMEM((GRP * D, F), jnp.float32),
            pltpu.VMEM((GRP, F), jnp.float32),
            pltpu.VMEM((L,), jnp.float32),
            pltpu.SemaphoreType.DMA,
            pltpu.SemaphoreType.DMA,
        ],
    )(rfl, key, query, nh)


def _sc3_body(n1, rf_hbm, key_hbm, q_hbm, adj_hbm, out_hbm,
              rf_v, key_v, nbr_v, nq_v, oid_v, sem_a, sem_q):
    wid = lax.axis_index("s") * NC + lax.axis_index("c")
    base = wid * RPW
    pltpu.sync_copy(rf_hbm.at[pl.ds(base, RPW)], rf_v)
    pltpu.sync_copy(key_hbm.at[pl.ds(base, RPW)], key_v)

    @pl.loop(0, RPW)
    def _row(r):
        pltpu.async_copy(adj_hbm.at[rf_v.at[r]], nbr_v, sem_a).wait()
        cps = [pltpu.async_copy(q_hbm.at[nbr_v.at[j]],
                                nq_v.at[pl.ds(j * D, D)], sem_q)
               for j in range(D)]
        for cp in cps:
            cp.wait()
        kb = [jnp.full((L,), key_v[r, l]) for l in range(ATT)]
        tv = jnp.full((L,), NEG)
        tid = jnp.zeros((L,), jnp.int32)
        for j in range(D):
            m16 = j * D + _iota16()
            a = jnp.zeros((L,), jnp.float32)
            for l in range(ATT):
                col = plsc.load_gather(nq_v,
                                       [m16, jnp.full((L,), l, jnp.int32)])
                a = a + col * kb[l]
            ids = nbr_v[j, :]
            a = jnp.where(ids == n1, NEG, a)
            sv, sid = plsc.sort_key_val(a, ids, descending=True)
            if j == 0:
                tv, tid = sv, sid
            else:
                rv = lax.rev(sv, (0,))
                rid = lax.rev(sid, (0,))
                mv = jnp.maximum(tv, rv)
                mid = jnp.where(tv >= rv, tid, rid)
                tv, tid = plsc.sort_key_val(mv, mid, descending=True)
        oid_v[r, :] = tid

    pltpu.sync_copy(oid_v, out_hbm.at[pl.ds(base, RPW)])


def _sc_stage3(rf_pad, key, query, adj, n1):
    mesh = plsc.VectorSubcoreMesh(core_axis_name="c", subcore_axis_name="s")
    return pl.kernel(
        functools.partial(_sc3_body, n1),
        out_type=jax.ShapeDtypeStruct((NPAD, D), jnp.int32),
        mesh=mesh,
        scratch_types=[
            pltpu.VMEM((RPW, D), jnp.int32),
            pltpu.VMEM((RPW, ATT), jnp.float32),
            pltpu.VMEM((D, D), jnp.int32),
            pltpu.VMEM((D * D, ATT), jnp.float32),
            pltpu.VMEM((RPW, D), jnp.int32),
            pltpu.SemaphoreType.DMA,
            pltpu.SemaphoreType.DMA,
        ],
    )(rf_pad, key, query, adj)


def kernel(input, receptive_field, adj, W, Wk, Wq, bn_gamma, bn_beta):
    n = input.shape[0]
    n1 = n - 1
    x_pad = jnp.pad(input, ((0, NPAD - n), (0, 0)))
    wkq = jnp.concatenate([Wk, Wq], axis=1)
    kq, nh = _matmuls(x_pad, wkq, W[0])
    key = kq[:, :ATT]
    query = kq[:, ATT:]

    rf1 = receptive_field[0]
    rf_pad = jnp.pad(rf1, ((0, NPAD - n), (0, 0)))
    rfl = rf_pad.reshape(-1)

    fh0 = _sc_stage1(rfl, key, query, nh, n1)
    fh = _batchnorm_relu(fh0, bn_gamma, bn_beta, n)[:n]

    expand = _sc_stage3(rf_pad, key, query, adj, n1)[:n]
    rf_new = jnp.concatenate([receptive_field, expand[None]], axis=0)
    return fh, rf_new


# speedup vs baseline: 20.8014x; 20.8014x over previous
import functools

import jax
import jax.numpy as jnp
from jax import lax
from jax.experimental import pallas as pl
from jax.experimental.pallas import tpu as pltpu
from jax.experimental.pallas import tpu_sc as plsc

L = 16
NC = 2
NS = 16
NW = NC * NS
RPW = 320
NPAD = NW * RPW
D = 16
ATT = 16
F = 128
NEG = -1e30

_SC_PARAMS = pltpu.CompilerParams(
    needs_layout_passes=False, use_tc_tiling_on_sc=False)


def _iota16():
    return lax.iota(jnp.int32, L)


def _scmesh():
    return plsc.VectorSubcoreMesh(core_axis_name="c", subcore_axis_name="s")


def _mm_body(x_ref, wkq_ref, w_ref, kq_ref, nh_ref):
    x = x_ref[...]
    kq_ref[...] = jnp.dot(x, wkq_ref[...], preferred_element_type=jnp.float32)
    nh_ref[...] = jnp.dot(x, w_ref[...], preferred_element_type=jnp.float32)


def _matmuls(x_pad, wkq, w):
    blk = 512
    grid = NPAD // blk
    return pl.pallas_call(
        _mm_body,
        grid=(grid,),
        in_specs=[
            pl.BlockSpec((blk, F), lambda i: (i, 0)),
            pl.BlockSpec((F, 2 * ATT), lambda i: (0, 0)),
            pl.BlockSpec((F, F), lambda i: (0, 0)),
        ],
        out_specs=[
            pl.BlockSpec((blk, 2 * ATT), lambda i: (i, 0)),
            pl.BlockSpec((blk, F), lambda i: (i, 0)),
        ],
        out_shape=[
            jax.ShapeDtypeStruct((NPAD, 2 * ATT), jnp.float32),
            jax.ShapeDtypeStruct((NPAD, F), jnp.float32),
        ],
    )(x_pad, wkq, w)


def _bn_body(n_valid, x_ref, g_ref, b_ref, o_ref):
    x = x_ref[...]
    rows = lax.broadcasted_iota(jnp.int32, x.shape, 0)
    xm = jnp.where(rows < n_valid, x, 0.0)
    s = jnp.sum(xm, axis=0, keepdims=True)
    ss = jnp.sum(xm * xm, axis=0, keepdims=True)
    mean = s / n_valid
    var = ss / n_valid - mean * mean
    inv = lax.rsqrt(var + 1e-5)
    y = g_ref[...] * (x - mean) * inv + b_ref[...]
    o_ref[...] = jnp.maximum(y, 0.0)


def _batchnorm_relu(fh0, gamma, beta, n_valid):
    return pl.pallas_call(
        functools.partial(_bn_body, n_valid),
        out_shape=jax.ShapeDtypeStruct((NPAD, F), jnp.float32),
    )(fh0, gamma.reshape(1, F), beta.reshape(1, F))


GRP = 8
NGRP = RPW // GRP


def _sc1_body(n1, rfl_hbm, key_hbm, q_hbm, nh_hbm, out_hbm,
              rfl_v, key_v, nhs_v, qg0, qg1, nhg0, nhg1, og,
              sq0, sq1, sn0, sn1):
    wid = lax.axis_index("s") * NC + lax.axis_index("c")
    base = wid * RPW
    pltpu.sync_copy(rfl_hbm.at[pl.ds(base * D, RPW * D)], rfl_v)
    pltpu.sync_copy(key_hbm.at[pl.ds(base, RPW)], key_v)
    pltpu.sync_copy(nh_hbm.at[pl.ds(base, RPW)], nhs_v)

    qgs, nhgs, sqs, sns = (qg0, qg1), (nhg0, nhg1), (sq0, sq1), (sn0, sn1)

    def _issue(g, b):
        idx = rfl_v.at[pl.ds(g * (GRP * D), GRP * D)]
        pltpu.async_copy(q_hbm.at[idx], qgs[b], sqs[b])
        pltpu.async_copy(nh_hbm.at[idx], nhgs[b], sns[b])

    def _wait(g, b):
        idx = rfl_v.at[pl.ds(g * (GRP * D), GRP * D)]
        pltpu.make_async_copy(q_hbm.at[idx], qgs[b], sqs[b]).wait()
        pltpu.make_async_copy(nh_hbm.at[idx], nhgs[b], sns[b]).wait()

    _issue(0, 0)
    _issue(1, 1)

    @pl.loop(0, NGRP, step=2)
    def _group(g):
        for b in range(2):
            cur = g + b
            _wait(cur, b)
            qg, nhg = qgs[b], nhgs[b]

            @pl.loop(0, GRP)
            def _row(r8):
                row = cur * GRP + r8
                kvec = key_v[row, :]
                kb = [jnp.full((L,), kvec[l]) for l in range(ATT)]
                m16 = r8 * D + _iota16()
                recep = jnp.zeros((L,), jnp.float32)
                for l in range(ATT):
                    col = plsc.load_gather(
                        qg, [m16, jnp.full((L,), l, jnp.int32)])
                    recep = recep + col * kb[l]
                rfrow = rfl_v[pl.ds(row * D, D)]
                recep = jnp.where(rfrow == n1, NEG, recep)
                mx = jnp.max(recep)
                e = jnp.exp(recep - mx)
                att = e / jnp.sum(e)
                acc = [nhs_v[row, pl.ds(c * L, L)] for c in range(F // L)]
                for j in range(D):
                    wb = jnp.full((L,), att[j])
                    for c in range(F // L):
                        acc[c] = acc[c] + wb * nhg[r8 * D + j, pl.ds(c * L, L)]
                for c in range(F // L):
                    og[r8, pl.ds(c * L, L)] = acc[c]

            pltpu.sync_copy(og, out_hbm.at[pl.ds(base + cur * GRP, GRP)])

            @pl.when(cur + 2 < NGRP)
            def _():
                _issue(cur + 2, b)


def _sc_stage1(rfl, key, query, nh, n1):
    return pl.kernel(
        functools.partial(_sc1_body, n1),
        out_type=jax.ShapeDtypeStruct((NPAD, F), jnp.float32),
        mesh=_scmesh(),
        compiler_params=_SC_PARAMS,
        scratch_types=[
            pltpu.VMEM((RPW * D,), jnp.int32),
            pltpu.VMEM((RPW, ATT), jnp.float32),
            pltpu.VMEM((RPW, F), jnp.float32),
            pltpu.VMEM((GRP * D, ATT), jnp.float32),
            pltpu.VMEM((GRP * D, ATT), jnp.float32),
            pltpu.VMEM((GRP * D, F), jnp.float32),
            pltpu.VMEM((GRP * D, F), jnp.float32),
            pltpu.VMEM((GRP, F), jnp.float32),
            pltpu.SemaphoreType.DMA,
            pltpu.SemaphoreType.DMA,
            pltpu.SemaphoreType.DMA,
            pltpu.SemaphoreType.DMA,
        ],
    )(rfl, key, query, nh)


def _gb_body(adjf_hbm, q_hbm, g_hbm, adjf_v, qa0, qa1, gt, s0, s1):
    wid = lax.axis_index("s") * NC + lax.axis_index("c")
    base = wid * RPW
    pltpu.sync_copy(adjf_hbm.at[pl.ds(base * D, RPW * D)], adjf_v)
    qas, ss = (qa0, qa1), (s0, s1)

    def _issue(g, b):
        idx = adjf_v.at[pl.ds(g * (GRP * D), GRP * D)]
        pltpu.async_copy(q_hbm.at[idx], qas[b], ss[b])

    def _wait(g, b):
        idx = adjf_v.at[pl.ds(g * (GRP * D), GRP * D)]
        pltpu.make_async_copy(q_hbm.at[idx], qas[b], ss[b]).wait()

    _issue(0, 0)
    _issue(1, 1)

    @pl.loop(0, NGRP, step=2)
    def _group(g):
        for b in range(2):
            cur = g + b
            _wait(cur, b)
            qa = qas[b]

            @pl.loop(0, GRP)
            def _node(r8):
                m16 = r8 * D + _iota16()
                for l in range(ATT):
                    col = plsc.load_gather(
                        qa, [m16, jnp.full((L,), l, jnp.int32)])
                    gt[r8, pl.ds(l * L, L)] = col

            pltpu.sync_copy(gt, g_hbm.at[pl.ds(base + cur * GRP, GRP)])

            @pl.when(cur + 2 < NGRP)
            def _():
                _issue(cur + 2, b)


def _g_build(adjf, query):
    return pl.kernel(
        _gb_body,
        out_type=jax.ShapeDtypeStruct((NPAD, D * ATT), jnp.float32),
        mesh=_scmesh(),
        compiler_params=_SC_PARAMS,
        scratch_types=[
            pltpu.VMEM((RPW * D,), jnp.int32),
            pltpu.VMEM((GRP * D, ATT), jnp.float32),
            pltpu.VMEM((GRP * D, ATT), jnp.float32),
            pltpu.VMEM((GRP, D * ATT), jnp.float32),
            pltpu.SemaphoreType.DMA,
            pltpu.SemaphoreType.DMA,
        ],
    )(adjf, query)


def _merge16(av, ai, bv, bi):
    rv = lax.rev(bv, (0,))
    ri = lax.rev(bi, (0,))
    mv = jnp.maximum(av, rv)
    mi = jnp.where(av >= rv, ai, ri)
    return plsc.sort_key_val(mv, mi, descending=True)


def _sc3_body(n1, rf_hbm, key_hbm, g_hbm, adj_hbm, out_hbm,
              rf_v, key_v, nb0, nb1, gr0, gr1, oid_v, s0, s1):
    wid = lax.axis_index("s") * NC + lax.axis_index("c")
    base = wid * RPW
    pltpu.sync_copy(rf_hbm.at[pl.ds(base, RPW)], rf_v)
    pltpu.sync_copy(key_hbm.at[pl.ds(base, RPW)], key_v)
    nbs, grs, ss = (nb0, nb1), (gr0, gr1), (s0, s1)

    def _issue(r, b):
        idx = rf_v.at[r]
        pltpu.async_copy(adj_hbm.at[idx], nbs[b], ss[b])
        pltpu.async_copy(g_hbm.at[idx], grs[b], ss[b])

    def _wait(r, b):
        idx = rf_v.at[r]
        pltpu.make_async_copy(adj_hbm.at[idx], nbs[b], ss[b]).wait()
        pltpu.make_async_copy(g_hbm.at[idx], grs[b], ss[b]).wait()

    _issue(0, 0)
    _issue(1, 1)

    @pl.loop(0, RPW, step=2)
    def _rows(r):
        for b in range(2):
            cur = r + b
            _wait(cur, b)
            nbr, gr = nbs[b], grs[b]
            kvec = key_v[cur, :]
            kb = [jnp.full((L,), kvec[l]) for l in range(ATT)]
            pairs = []
            for j in range(D):
                a = jnp.zeros((L,), jnp.float32)
                for l in range(ATT):
                    a = a + kb[l] * gr[j, pl.ds(l * L, L)]
                ids = nbr[j, :]
                a = jnp.where(ids == n1, NEG, a)
                pairs.append(plsc.sort_key_val(a, ids, descending=True))

            @pl.when(cur + 2 < RPW)
            def _():
                _issue(cur + 2, b)

            while len(pairs) > 1:
                nxt = []
                for i in range(0, len(pairs), 2):
                    nxt.append(_merge16(pairs[i][0], pairs[i][1],
                                        pairs[i + 1][0], pairs[i + 1][1]))
                pairs = nxt
            oid_v[cur, :] = pairs[0][1]

    pltpu.sync_copy(oid_v, out_hbm.at[pl.ds(base, RPW)])


def _sc_stage3(rf_pad, key, g_tab, adj, n1):
    return pl.kernel(
        functools.partial(_sc3_body, n1),
        out_type=jax.ShapeDtypeStruct((NPAD, D), jnp.int32),
        mesh=_scmesh(),
        compiler_params=_SC_PARAMS,
        scratch_types=[
            pltpu.VMEM((RPW, D), jnp.int32),
            pltpu.VMEM((RPW, ATT), jnp.float32),
            pltpu.VMEM((D, D), jnp.int32),
            pltpu.VMEM((D, D), jnp.int32),
            pltpu.VMEM((D, D * ATT), jnp.float32),
            pltpu.VMEM((D, D * ATT), jnp.float32),
            pltpu.VMEM((RPW, D), jnp.int32),
            pltpu.SemaphoreType.DMA,
            pltpu.SemaphoreType.DMA,
        ],
    )(rf_pad, key, g_tab, adj)


def kernel(input, receptive_field, adj, W, Wk, Wq, bn_gamma, bn_beta):
    n = input.shape[0]
    n1 = n - 1
    x_pad = jnp.pad(input, ((0, NPAD - n), (0, 0)))
    wkq = jnp.concatenate([Wk, Wq], axis=1)
    kq, nh = _matmuls(x_pad, wkq, W[0])
    key = kq[:, :ATT]
    query = kq[:, ATT:]

    rf1 = receptive_field[0]
    rf_pad = jnp.pad(rf1, ((0, NPAD - n), (0, 0)))
    rfl = rf_pad.reshape(-1)
    adjf = jnp.pad(adj, ((0, NPAD - n), (0, 0))).reshape(-1)

    fh0 = _sc_stage1(rfl, key, query, nh, n1)
    fh = _batchnorm_relu(fh0, bn_gamma, bn_beta, n)[:n]

    g_tab = _g_build(adjf, query)
    expand = _sc_stage3(rf_pad, key, g_tab, adj, n1)[:n]
    rf_new = jnp.concatenate([receptive_field, expand[None]], axis=0)
    return fh, rf_new
